# baseline (speedup 1.0000x reference)
import dataclasses
import functools

import jax
import jax.numpy as jnp
from jax import lax
from jax.experimental import pallas as pl
from jax.experimental.pallas import tpu as pltpu
from jax.experimental.pallas import tpu_sc as plsc

ALPHA = 0.3
TARGET = 729
K_SEL = TARGET - 1
N_TOK = 3645
N_PAD = 4096
CHUNK = 128
N_CHUNK = N_PAD // CHUNK
D = 1152
L_TXT = 256
N_DISCARD = N_TOK - K_SEL
RW = 24
NWF = 30
TAILR = K_SEL - NWF * RW
NW = NWF + 1
IDXW = NW * CHUNK


def _f32_key(bits):
    return jnp.where(bits >= 0, bits, bits ^ jnp.int32(0x7FFFFFFF))


RT = 512
NT = N_PAD // RT


def _sal_body(img_ref, txt_ref, sal_ref):
    t = pl.program_id(1)
    img = img_ref[0]
    txt = txt_ref[0]

    tnorm = jnp.sqrt(jnp.sum(txt * txt, axis=1, keepdims=True))
    txtn = txt / jnp.maximum(tnorm, 1e-12)

    s1 = jnp.sum(img, axis=1, keepdims=True)
    mean = s1 / D
    cen = img - mean
    var = jnp.sum(cen * cen, axis=1) / (D - 1)

    inorm = jnp.sqrt(jnp.sum(img * img, axis=1, keepdims=True))
    imgn = img / jnp.maximum(inorm, 1e-12)
    sim = lax.dot_general(imgn.astype(jnp.bfloat16), txtn.astype(jnp.bfloat16),
                          (((1,), (1,)), ((), ())),
                          preferred_element_type=jnp.float32,
                          precision=lax.Precision.DEFAULT)
    rel = jnp.max(sim, axis=1)
    sal = ALPHA * var + (1.0 - ALPHA) * rel
    gidx = lax.broadcasted_iota(jnp.int32, (1, RT), 1)[0] + t * RT
    sal_ref[0, 0, :] = jnp.where(gidx < N_TOK, sal, -jnp.inf)


def _select_body(salf_ref, salt_ref, img_ref, idx_ref, summ_ref,
                 st_ref, acc_ref):
    t = pl.program_id(1)
    bprog = pl.program_id(0)

    @pl.when(t == 0)
    def _bisect():
        sal_pad = salf_ref[0, 0, :]
        idxv = lax.broadcasted_iota(jnp.int32, (1, N_PAD), 1)[0]
        key = _f32_key(lax.bitcast_convert_type(sal_pad, jnp.int32))

        def bit_step(i, prefix):
            cand = prefix + jnp.left_shift(jnp.int32(1), 30 - i)
            cnt = jnp.sum(jnp.where(key >= cand, 1, 0))
            return jnp.where(cnt >= K_SEL, cand, prefix)

        nonneg = jnp.sum(jnp.where(key >= 0, 1, 0))
        start = jnp.where(nonneg >= K_SEL, jnp.int32(0),
                          jnp.int32(-2147483648))
        tkey = lax.fori_loop(0, 31, bit_step, start)

        m = jnp.sum(jnp.where(key > tkey, 1, 0))
        tt = K_SEL - m
        iseq = key == tkey

        def tie_step(i, cpre):
            cand = cpre | jnp.left_shift(jnp.int32(1), 11 - i)
            cnt = jnp.sum(jnp.where(iseq & (idxv < cand), 1, 0))
            return jnp.where(cnt < tt, cand, cpre)

        cidx = lax.fori_loop(0, 12, tie_step, jnp.int32(0))

        sel = (key > tkey) | (iseq & (idxv <= cidx))
        maskf = jnp.where(sel, 1.0, 0.0).astype(jnp.float32)

        csum = maskf
        for p in range(12):
            sh = 1 << p
            csum = csum + jnp.concatenate(
                [jnp.zeros((sh,), jnp.float32), csum[:N_PAD - sh]])
        pos = csum - 1.0

        qlane = lax.broadcasted_iota(jnp.int32, (1, K_SEL), 1).astype(
            jnp.float32)
        posm = jnp.where(sel, pos, -2.0)
        onehot = jnp.where(posm[:, None] == qlane, 1.0, 0.0).astype(
            jnp.float32)
        ivec = (lax.broadcasted_iota(jnp.int32, (1, N_PAD), 1)
                + N_TOK * bprog).astype(jnp.float32)
        idxf = lax.dot_general(ivec, onehot, (((1,), (0,)), ((), ())),
                               preferred_element_type=jnp.float32,
                               precision=lax.Precision.HIGHEST)
        idxi = idxf[0].astype(jnp.int32)
        zpad = jnp.zeros((CHUNK - RW,), jnp.int32)
        parts = []
        for wk in range(NWF):
            parts.append(idxi[wk * RW:(wk + 1) * RW])
            parts.append(zpad)
        parts.append(idxi[NWF * RW:K_SEL])
        parts.append(jnp.zeros((CHUNK - TAILR,), jnp.int32))
        idx_ref[0, 0, :] = jnp.concatenate(parts)
        st_ref[0] = tkey
        st_ref[1] = cidx

    tkey = st_ref[0]
    cidx = st_ref[1]
    sal_t = salt_ref[0, 0, :]
    gidx = lax.broadcasted_iota(jnp.int32, (1, RT), 1)[0] + t * RT
    key_t = _f32_key(lax.bitcast_convert_type(sal_t, jnp.int32))
    sel_t = (key_t > tkey) | ((key_t == tkey) & (gidx <= cidx))
    umask = jnp.where(sel_t | (gidx >= N_TOK), 0.0, 1.0).astype(jnp.float32)
    gcol = lax.broadcasted_iota(jnp.int32, (RT, 1), 0) + t * RT
    imgz = jnp.where(gcol < N_TOK, img_ref[0], 0.0)
    part = lax.dot_general(umask[None, :], imgz,
                           (((1,), (0,)), ((), ())),
                           preferred_element_type=jnp.float32,
                           precision=lax.Precision.HIGHEST)

    @pl.when(t == 0)
    def _init():
        acc_ref[...] = part

    @pl.when(t > 0)
    def _accum():
        acc_ref[...] += part

    @pl.when(t == NT - 1)
    def _finish():
        summ_ref[0, 0, :] = acc_ref[0] / float(N_DISCARD)


def _tc_saliency(img, txt):
    B = img.shape[0]
    return pl.pallas_call(
        _sal_body,
        grid=(B, NT),
        in_specs=[
            pl.BlockSpec((1, RT, D), lambda b, t: (b, t, 0)),
            pl.BlockSpec((1, L_TXT, D), lambda b, t: (b, 0, 0)),
        ],
        out_specs=pl.BlockSpec((1, 1, RT), lambda b, t: (b, 0, t)),
        out_shape=jax.ShapeDtypeStruct((B, 1, N_PAD), jnp.float32),
    )(img, txt)


def _tc_select(sal_pad, img):
    B = img.shape[0]
    return pl.pallas_call(
        _select_body,
        grid=(B, NT),
        in_specs=[
            pl.BlockSpec((1, 1, N_PAD), lambda b, t: (b, 0, 0)),
            pl.BlockSpec((1, 1, RT), lambda b, t: (b, 0, t)),
            pl.BlockSpec((1, RT, D), lambda b, t: (b, t, 0)),
        ],
        out_specs=[
            pl.BlockSpec((1, 1, IDXW), lambda b, t: (b, 0, 0)),
            pl.BlockSpec((1, 1, D), lambda b, t: (b, 0, 0)),
        ],
        out_shape=[
            jax.ShapeDtypeStruct((B, 1, IDXW), jnp.int32),
            jax.ShapeDtypeStruct((B, 1, D), jnp.float32),
        ],
        scratch_shapes=[
            pltpu.SMEM((2,), jnp.int32),
            pltpu.VMEM((1, D), jnp.float32),
        ],
    )(sal_pad, sal_pad, img)


def _sc_select(img_flat, idx, summ, B):
    mesh = plsc.VectorSubcoreMesh(core_axis_name="c", subcore_axis_name="s")
    cp = pltpu.CompilerParams()
    if "needs_layout_passes" in pltpu.CompilerParams.__dataclass_fields__:
        cp = dataclasses.replace(cp, needs_layout_passes=False)

    @functools.partial(
        pl.kernel,
        mesh=mesh,
        compiler_params=cp,
        out_type=jax.ShapeDtypeStruct((B, TARGET, D), jnp.float32),
        scratch_types=[
            pltpu.VMEM((1, CHUNK), jnp.int32),
            pltpu.VMEM((RW, D), jnp.float32),
            pltpu.VMEM((TAILR, D), jnp.float32),
            pltpu.VMEM((1, D), jnp.float32),
            pltpu.SemaphoreType.DMA,
        ],
    )
    def sc_kernel(img_hbm, idx_hbm, summ_hbm, out_hbm, idxv, stage, stage8,
                  sumv, sem):
        cid = lax.axis_index("c")
        sid = lax.axis_index("s")
        w = sid * 2 + cid

        @pl.when(w < NWF)
        def _gather():
            for b in range(B):
                pltpu.sync_copy(idx_hbm.at[b, w], idxv)
                myidx = idxv.at[0, pl.ds(0, RW)]
                pltpu.async_copy(img_hbm.at[myidx], stage, sem).wait()
                pltpu.sync_copy(stage, out_hbm.at[b, pl.ds(w * RW, RW)])

        @pl.when(w == NWF)
        def _tail():
            for b in range(B):
                pltpu.sync_copy(idx_hbm.at[b, NWF], idxv)
                myidx = idxv.at[0, pl.ds(0, TAILR)]
                pltpu.async_copy(img_hbm.at[myidx], stage8, sem).wait()
                pltpu.sync_copy(stage8,
                                out_hbm.at[b, pl.ds(NWF * RW, TAILR)])

        @pl.when(w == NW)
        def _summaries():
            for b in range(B):
                pltpu.sync_copy(summ_hbm.at[b], sumv)
                pltpu.sync_copy(sumv, out_hbm.at[b, pl.ds(K_SEL, 1)])

    return sc_kernel(img_flat, idx, summ)


def kernel(image_tokens, text_embeddings):
    B = image_tokens.shape[0]
    sal_pad = _tc_saliency(image_tokens, text_embeddings)
    idx, summ = _tc_select(sal_pad, image_tokens)
    return _sc_select(image_tokens.reshape(B * N_TOK, D),
                      idx.reshape(B, NW, 1, CHUNK),
                      summ.reshape(B, 1, D), B)


# speedup vs baseline: 1.0274x; 1.0274x over previous
import dataclasses
import functools

import jax
import jax.numpy as jnp
from jax import lax
from jax.experimental import pallas as pl
from jax.experimental.pallas import tpu as pltpu
from jax.experimental.pallas import tpu_sc as plsc

ALPHA = 0.3
TARGET = 729
K_SEL = TARGET - 1
N_TOK = 3645
N_PAD = 4096
CHUNK = 128
D = 1152
L_TXT = 256
N_DISCARD = N_TOK - K_SEL
RW = 48
NSF = 15
TAILR = K_SEL - NSF * RW
NSUB = 16
IDXW = NSUB * CHUNK

RT = 512
NT = N_PAD // RT


def _f32_key(bits):
    return jnp.where(bits >= 0, bits, bits ^ jnp.int32(0x7FFFFFFF))


def _sal_body(img_ref, txt_ref, sal_ref, csum_ref):
    t = pl.program_id(1)
    img = img_ref[0]
    txt = txt_ref[0]

    tnorm = jnp.sqrt(jnp.sum(txt * txt, axis=1, keepdims=True))
    txtn = txt / jnp.maximum(tnorm, 1e-12)

    s1 = jnp.sum(img, axis=1, keepdims=True)
    mean = s1 / D
    cen = img - mean
    var = jnp.sum(cen * cen, axis=1) / (D - 1)

    inorm = jnp.sqrt(jnp.sum(img * img, axis=1, keepdims=True))
    imgn = img / jnp.maximum(inorm, 1e-12)
    sim = lax.dot_general(imgn.astype(jnp.bfloat16), txtn.astype(jnp.bfloat16),
                          (((1,), (1,)), ((), ())),
                          preferred_element_type=jnp.float32,
                          precision=lax.Precision.DEFAULT)
    rel = jnp.max(sim, axis=1)
    sal = ALPHA * var + (1.0 - ALPHA) * rel
    gidx = lax.broadcasted_iota(jnp.int32, (1, RT), 1)[0] + t * RT
    sal_ref[0, 0, :] = jnp.where(gidx < N_TOK, sal, -jnp.inf)

    gcol = lax.broadcasted_iota(jnp.int32, (RT, 1), 0) + t * RT
    imgz = jnp.where(gcol < N_TOK, img, 0.0)
    part = jnp.sum(imgz, axis=0)

    @pl.when(t == 0)
    def _init():
        csum_ref[0, 0, :] = part

    @pl.when(t > 0)
    def _accum():
        csum_ref[0, 0, :] += part


def _tc_saliency(img, txt):
    B = img.shape[0]
    return pl.pallas_call(
        _sal_body,
        grid=(B, NT),
        in_specs=[
            pl.BlockSpec((1, RT, D), lambda b, t: (b, t, 0)),
            pl.BlockSpec((1, L_TXT, D), lambda b, t: (b, 0, 0)),
        ],
        out_specs=[
            pl.BlockSpec((1, 1, RT), lambda b, t: (b, 0, t)),
            pl.BlockSpec((1, 1, D), lambda b, t: (b, 0, 0)),
        ],
        out_shape=[
            jax.ShapeDtypeStruct((B, 1, N_PAD), jnp.float32),
            jax.ShapeDtypeStruct((B, 1, D), jnp.float32),
        ],
    )(img, txt)


def _select_body(salf_ref, idx_ref):
    bprog = pl.program_id(0)
    sal_pad = salf_ref[0, 0, :]
    idxv = lax.broadcasted_iota(jnp.int32, (1, N_PAD), 1)[0]
    key = _f32_key(lax.bitcast_convert_type(sal_pad, jnp.int32))

    def bit_step(i, prefix):
        cand = prefix + jnp.left_shift(jnp.int32(1), 30 - i)
        cnt = jnp.sum(jnp.where(key >= cand, 1, 0))
        return jnp.where(cnt >= K_SEL, cand, prefix)

    nonneg = jnp.sum(jnp.where(key >= 0, 1, 0))
    start = jnp.where(nonneg >= K_SEL, jnp.int32(0), jnp.int32(-2147483648))
    tkey = lax.fori_loop(0, 31, bit_step, start)

    m = jnp.sum(jnp.where(key > tkey, 1, 0))
    tt = K_SEL - m
    iseq = key == tkey

    def tie_step(i, cpre):
        cand = cpre | jnp.left_shift(jnp.int32(1), 11 - i)
        cnt = jnp.sum(jnp.where(iseq & (idxv < cand), 1, 0))
        return jnp.where(cnt < tt, cand, cpre)

    cidx = lax.fori_loop(0, 12, tie_step, jnp.int32(0))

    sel = (key > tkey) | (iseq & (idxv <= cidx))
    maskf = jnp.where(sel, 1.0, 0.0).astype(jnp.float32)

    csum = maskf
    for p in range(12):
        sh = 1 << p
        csum = csum + jnp.concatenate(
            [jnp.zeros((sh,), jnp.float32), csum[:N_PAD - sh]])
    pos = csum - 1.0

    qlane = lax.broadcasted_iota(jnp.int32, (1, K_SEL), 1).astype(jnp.float32)
    posm = jnp.where(sel, pos, -2.0)
    onehot = jnp.where(posm[:, None] == qlane, 1.0, 0.0).astype(jnp.float32)
    ivec = (lax.broadcasted_iota(jnp.int32, (1, N_PAD), 1)
            + N_TOK * bprog).astype(jnp.float32)
    idxf = lax.dot_general(ivec, onehot, (((1,), (0,)), ((), ())),
                           preferred_element_type=jnp.float32,
                           precision=lax.Precision.HIGHEST)
    idxi = idxf[0].astype(jnp.int32)
    zpad = jnp.zeros((CHUNK - RW,), jnp.int32)
    parts = []
    for wk in range(NSF):
        parts.append(idxi[wk * RW:(wk + 1) * RW])
        parts.append(zpad)
    parts.append(idxi[NSF * RW:K_SEL])
    parts.append(jnp.zeros((CHUNK - TAILR,), jnp.int32))
    idx_ref[0, 0, :] = jnp.concatenate(parts)


def _tc_select(sal_pad):
    B = sal_pad.shape[0]
    return pl.pallas_call(
        _select_body,
        grid=(B,),
        in_specs=[pl.BlockSpec((1, 1, N_PAD), lambda b: (b, 0, 0))],
        out_specs=pl.BlockSpec((1, 1, IDXW), lambda b: (b, 0, 0)),
        out_shape=jax.ShapeDtypeStruct((B, 1, IDXW), jnp.int32),
    )(sal_pad)


def _sc_select(img_flat, idx, csum, B):
    mesh = plsc.VectorSubcoreMesh(core_axis_name="c", subcore_axis_name="s")
    cp = pltpu.CompilerParams()
    if "needs_layout_passes" in pltpu.CompilerParams.__dataclass_fields__:
        cp = dataclasses.replace(cp, needs_layout_passes=False)

    @functools.partial(
        pl.kernel,
        mesh=mesh,
        compiler_params=cp,
        out_type=jax.ShapeDtypeStruct((B, TARGET, D), jnp.float32),
        scratch_types=[
            pltpu.VMEM((1, CHUNK), jnp.int32),
            pltpu.VMEM((RW, D), jnp.float32),
            pltpu.VMEM((TAILR, D), jnp.float32),
            pltpu.VMEM((1, D), jnp.float32),
            pltpu.VMEM((1, D), jnp.float32),
            pltpu.VMEM((2 * NSUB, D), jnp.float32),
            pltpu.VMEM_SHARED((2 * NSUB, D), jnp.float32),
            pltpu.SemaphoreType.DMA,
        ],
    )
    def sc_kernel(img_hbm, idx_hbm, csum_hbm, out_hbm, idxv, stage, stage8,
                  psum, zrow, pread, shared, sem):
        c = lax.axis_index("c")
        s = lax.axis_index("s")

        for bi in range(2):
            b = c * 2 + bi
            pltpu.sync_copy(idx_hbm.at[b, s], idxv)
            for k in range(D // 16):
                psum[0, pl.ds(k * 16, 16)] = jnp.zeros((16,), jnp.float32)

            @pl.when(s < NSF)
            def _full():
                myidx = idxv.at[0, pl.ds(0, RW)]
                pltpu.async_copy(img_hbm.at[myidx], stage, sem).wait()
                pltpu.sync_copy(stage, out_hbm.at[b, pl.ds(s * RW, RW)])

                @pl.loop(0, RW)
                def _rows(r):
                    for k in range(D // 16):
                        sl = pl.ds(k * 16, 16)
                        plsc.addupdate(psum.at[0, sl], stage[r, sl])

            @pl.when(s == NSF)
            def _tail():
                myidx = idxv.at[0, pl.ds(0, TAILR)]
                pltpu.async_copy(img_hbm.at[myidx], stage8, sem).wait()
                pltpu.sync_copy(stage8,
                                out_hbm.at[b, pl.ds(NSF * RW, TAILR)])

                @pl.loop(0, TAILR)
                def _rows8(r):
                    for k in range(D // 16):
                        sl = pl.ds(k * 16, 16)
                        plsc.addupdate(psum.at[0, sl], stage8[r, sl])

            pltpu.sync_copy(psum, shared.at[pl.ds(s * 2 + bi, 1)])

        plsc.subcore_barrier()

        @pl.when(s == NSF)
        def _summaries():
            pltpu.sync_copy(shared, pread)
            for bi in range(2):
                b = c * 2 + bi
                pltpu.sync_copy(csum_hbm.at[b], zrow)
                for k in range(D // 16):
                    sl = pl.ds(k * 16, 16)
                    acc = zrow[0, sl]
                    for r in range(NSUB):
                        acc = acc - pread[r * 2 + bi, sl]
                    zrow[0, sl] = acc * jnp.float32(1.0 / N_DISCARD)
                pltpu.sync_copy(zrow, out_hbm.at[b, pl.ds(K_SEL, 1)])

    return sc_kernel(img_flat, idx, csum)


def kernel(image_tokens, text_embeddings):
    B = image_tokens.shape[0]
    sal_pad, csum = _tc_saliency(image_tokens, text_embeddings)
    idx = _tc_select(sal_pad)
    return _sc_select(image_tokens.reshape(B * N_TOK, D),
                      idx.reshape(B, NSUB, 1, CHUNK),
                      csum.reshape(B, 1, D), B)
